# baseline (speedup 1.0000x reference)
import functools

import jax
import jax.numpy as jnp
from jax import lax
from jax.experimental import pallas as pl
from jax.experimental.pallas import tpu as pltpu
from jax.experimental.pallas import tpu_sc as plsc

MAX_LEN_ = 4096
D_MODEL_ = 1024
BATCH_ = 4
NC_ = 2
NS_ = 16
NW_ = NC_ * NS_
RPW_ = MAX_LEN_ // NW_
CROWS_ = 4
NCHUNK_ = RPW_ // CROWS_
LANES_ = 16
VPR_ = D_MODEL_ // LANES_
UNROLL_ = 8
NB_ = 5
LOOK_ = 2


def _sc_body(x_hbm, t_hbm, o_hbm,
             xb0, xb1, xb2, xb3, xb4, tb0, tb1, tb2,
             is0, is1, is2, is3, is4, os0, os1, os2, os3, os4,
             ts0, ts1, ts2):
    wid = lax.axis_index("s") * NC_ + lax.axis_index("c")
    wrow = wid * RPW_
    xbufs = (xb0, xb1, xb2, xb3, xb4)
    tbufs = (tb0, tb1, tb2)
    isems = (is0, is1, is2, is3, is4)
    osems = (os0, os1, os2, os3, os4)
    tsems = (ts0, ts1, ts2)

    def start_in(p):
        return pltpu.async_copy(
            x_hbm.at[:, pl.ds(wrow + p * CROWS_, CROWS_), :],
            xbufs[p % NB_], isems[p % NB_])

    def start_out(p):
        return pltpu.async_copy(
            xbufs[p % NB_],
            o_hbm.at[:, pl.ds(wrow + p * CROWS_, CROWS_), :], osems[p % NB_])

    def start_tbl(k):
        return pltpu.async_copy(
            t_hbm.at[pl.ds(wrow + k * CROWS_, CROWS_), :],
            tbufs[k % 3], tsems[k % 3])

    tbl_d = {k: start_tbl(k) for k in range(LOOK_)}
    in_d = {p: start_in(p) for p in range(LOOK_)}
    out_d = {}

    for p in range(NCHUNK_):
        if p + LOOK_ < NCHUNK_:
            if p + LOOK_ - NB_ >= 0:
                out_d[p + LOOK_ - NB_].wait()
            in_d[p + LOOK_] = start_in(p + LOOK_)
            tbl_d[p + LOOK_] = start_tbl(p + LOOK_)
        tbl_d[p].wait()
        in_d[p].wait()
        xbuf = xbufs[p % NB_]
        tbuf = tbufs[p % 3]

        for b in range(BATCH_):
            def add_vec(i, _b=b):
                r = i // VPR_
                c = (i % VPR_) * LANES_
                plsc.addupdate(xbuf.at[_b, r, pl.ds(c, LANES_)],
                               tbuf[r, pl.ds(c, LANES_)])

            plsc.parallel_loop(0, CROWS_ * VPR_, 1, unroll=UNROLL_)(add_vec)

        out_d[p] = start_out(p)

    for p in range(NCHUNK_ - NB_, NCHUNK_):
        out_d[p].wait()


_sc_add = functools.partial(
    pl.kernel,
    out_type=jax.ShapeDtypeStruct((BATCH_, MAX_LEN_, D_MODEL_), jnp.float32),
    mesh=plsc.VectorSubcoreMesh(core_axis_name="c", subcore_axis_name="s"),
    scratch_types=(
        [pltpu.VMEM((BATCH_, CROWS_, D_MODEL_), jnp.float32)] * NB_
        + [pltpu.VMEM((CROWS_, D_MODEL_), jnp.float32)] * 3
        + [pltpu.SemaphoreType.DMA] * (2 * NB_ + 3)
    ),
)(_sc_body)


def kernel(x, pos_table):
    return _sc_add(x, pos_table)


# speedup vs baseline: 1.0365x; 1.0365x over previous
import functools

import jax
import jax.numpy as jnp
from jax import lax
from jax.experimental import pallas as pl
from jax.experimental.pallas import tpu as pltpu
from jax.experimental.pallas import tpu_sc as plsc

MAX_LEN_ = 4096
D_MODEL_ = 1024
BATCH_ = 4
NC_ = 2
NS_ = 16
NW_ = NC_ * NS_
RPW_ = MAX_LEN_ // NW_
CROWS_ = 8
NCHUNK_ = RPW_ // CROWS_
LANES_ = 16
VPR_ = D_MODEL_ // LANES_
UNROLL_ = 8


def _sc_body(x_hbm, t_hbm, o_hbm,
             xb0, xb1, xb2, tb0, tb1,
             is0, is1, is2, os0, os1, os2, ts0, ts1):
    wid = lax.axis_index("s") * NC_ + lax.axis_index("c")
    wrow = wid * RPW_
    xbufs = (xb0, xb1, xb2)
    tbufs = (tb0, tb1)
    isems = (is0, is1, is2)
    osems = (os0, os1, os2)
    tsems = (ts0, ts1)

    def start_in(p):
        return pltpu.async_copy(
            x_hbm.at[:, pl.ds(wrow + p * CROWS_, CROWS_), :],
            xbufs[p % 3], isems[p % 3])

    def start_out(p):
        return pltpu.async_copy(
            xbufs[p % 3],
            o_hbm.at[:, pl.ds(wrow + p * CROWS_, CROWS_), :], osems[p % 3])

    def start_tbl(k):
        return pltpu.async_copy(
            t_hbm.at[pl.ds(wrow + k * CROWS_, CROWS_), :],
            tbufs[k % 2], tsems[k % 2])

    tbl_d = {0: start_tbl(0)}
    in_d = {0: start_in(0)}
    out_d = {}

    for p in range(NCHUNK_):
        if p + 1 < NCHUNK_:
            if p - 2 >= 0:
                out_d[p - 2].wait()
            in_d[p + 1] = start_in(p + 1)
            tbl_d[p + 1] = start_tbl(p + 1)
        tbl_d[p].wait()
        in_d[p].wait()
        xbuf = xbufs[p % 3]
        tbuf = tbufs[p % 2]

        for b in range(BATCH_):
            def add_vec(i, _b=b):
                r = i // VPR_
                c = (i % VPR_) * LANES_
                plsc.addupdate(xbuf.at[_b, r, pl.ds(c, LANES_)],
                               tbuf[r, pl.ds(c, LANES_)])

            plsc.parallel_loop(0, CROWS_ * VPR_, 1, unroll=UNROLL_)(add_vec)

        out_d[p] = start_out(p)

    for p in range(NCHUNK_ - 3, NCHUNK_):
        out_d[p].wait()


_sc_add = functools.partial(
    pl.kernel,
    out_type=jax.ShapeDtypeStruct((BATCH_, MAX_LEN_, D_MODEL_), jnp.float32),
    mesh=plsc.VectorSubcoreMesh(core_axis_name="c", subcore_axis_name="s"),
    scratch_types=(
        [pltpu.VMEM((BATCH_, CROWS_, D_MODEL_), jnp.float32)] * 3
        + [pltpu.VMEM((CROWS_, D_MODEL_), jnp.float32)] * 2
        + [pltpu.SemaphoreType.DMA] * 8
    ),
)(_sc_body)


def kernel(x, pos_table):
    return _sc_add(x, pos_table)


# speedup vs baseline: 1.0381x; 1.0015x over previous
import functools

import jax
import jax.numpy as jnp
from jax import lax
from jax.experimental import pallas as pl
from jax.experimental.pallas import tpu as pltpu
from jax.experimental.pallas import tpu_sc as plsc

MAX_LEN_ = 4096
D_MODEL_ = 1024
BATCH_ = 4
NC_ = 2
NS_ = 16
NW_ = NC_ * NS_
RPW_ = MAX_LEN_ // NW_
CROWS_ = 8
NCHUNK_ = RPW_ // CROWS_
LANES_ = 16
VPR_ = D_MODEL_ // LANES_
UNROLL_ = 8


def _sc_body(x_hbm, t_hbm, o_hbm,
             xb0, xb1, xb2, tb0, tb1,
             is0, is1, is2, os0, os1, os2, ts0, ts1):
    wid = lax.axis_index("s") * NC_ + lax.axis_index("c")
    wrow = wid * RPW_
    xbufs = (xb0, xb1, xb2)
    tbufs = (tb0, tb1)
    isems = (is0, is1, is2)
    osems = (os0, os1, os2)
    tsems = (ts0, ts1)

    def start_in(p):
        return pltpu.async_copy(
            x_hbm.at[:, pl.ds(wrow + p * CROWS_, CROWS_), :],
            xbufs[p % 3], isems[p % 3])

    def start_out(p, b):
        return pltpu.async_copy(
            xbufs[p % 3].at[b],
            o_hbm.at[b, pl.ds(wrow + p * CROWS_, CROWS_), :], osems[p % 3])

    def start_tbl(k):
        return pltpu.async_copy(
            t_hbm.at[pl.ds(wrow + k * CROWS_, CROWS_), :],
            tbufs[k % 2], tsems[k % 2])

    tbl_d = {0: start_tbl(0)}
    in_d = {0: start_in(0)}
    out_d = {}

    for p in range(NCHUNK_):
        if p + 1 < NCHUNK_:
            if p - 2 >= 0:
                for d in out_d[p - 2]:
                    d.wait()
            in_d[p + 1] = start_in(p + 1)
            tbl_d[p + 1] = start_tbl(p + 1)
        tbl_d[p].wait()
        in_d[p].wait()
        xbuf = xbufs[p % 3]
        tbuf = tbufs[p % 2]

        out_d[p] = []
        for b in range(BATCH_):
            def add_vec(i, _b=b):
                r = i // VPR_
                c = (i % VPR_) * LANES_
                plsc.addupdate(xbuf.at[_b, r, pl.ds(c, LANES_)],
                               tbuf[r, pl.ds(c, LANES_)])

            plsc.parallel_loop(0, CROWS_ * VPR_, 1, unroll=UNROLL_)(add_vec)
            out_d[p].append(start_out(p, b))

    for p in range(NCHUNK_ - 3, NCHUNK_):
        for d in out_d[p]:
            d.wait()


_sc_add = functools.partial(
    pl.kernel,
    out_type=jax.ShapeDtypeStruct((BATCH_, MAX_LEN_, D_MODEL_), jnp.float32),
    mesh=plsc.VectorSubcoreMesh(core_axis_name="c", subcore_axis_name="s"),
    scratch_types=(
        [pltpu.VMEM((BATCH_, CROWS_, D_MODEL_), jnp.float32)] * 3
        + [pltpu.VMEM((CROWS_, D_MODEL_), jnp.float32)] * 2
        + [pltpu.SemaphoreType.DMA] * 8
    ),
)(_sc_body)


def kernel(x, pos_table):
    return _sc_add(x, pos_table)
